# Initial kernel scaffold; baseline (speedup 1.0000x reference)
import jax
import jax.numpy as jnp
from jax.experimental import pallas as pl


def kernel(x_bcn, w1, w2, w3, w4, w5, bn1_scale, bn1_bias, bn2_scale, bn2_bias, bn3_scale, bn3_bias, bn4_scale, bn4_bias, bn5_scale, bn5_bias):
    raise NotImplementedError("write your pallas kernel here")


# speedup vs baseline: 2.8547x; 2.8547x over previous
import functools

import jax
import jax.numpy as jnp
from jax.experimental import pallas as pl
from jax.experimental.pallas import tpu as pltpu

_NEG_SLOPE = 0.2
_KNN = 20


def _pick_tile(n, target=256):
    if n <= target:
        return n
    for t in range(target, 7, -1):
        if n % t == 0 and t % 8 == 0:
            return t
    return n


def _leaky(v):
    return jnp.where(v > 0, v, _NEG_SLOPE * v)


def _pc_kernel(x_ref, w_ref, p_ref, c_ref):
    y = jnp.dot(x_ref[...], w_ref[...], preferred_element_type=jnp.float32)
    cout = p_ref.shape[-1]
    p_ref[...] = y[:, :cout]
    c_ref[...] = y[:, cout:]


def _point_conv(x, w_cat, tn):
    b, n, cin = x.shape
    cout = w_cat.shape[1] // 2
    return pl.pallas_call(
        _pc_kernel,
        out_shape=(
            jax.ShapeDtypeStruct((b, n, cout), jnp.float32),
            jax.ShapeDtypeStruct((b, n, cout), jnp.float32),
        ),
        grid=(b, n // tn),
        in_specs=[
            pl.BlockSpec((None, tn, cin), lambda bi, ni: (bi, ni, 0)),
            pl.BlockSpec((cin, 2 * cout), lambda bi, ni: (0, 0)),
        ],
        out_specs=(
            pl.BlockSpec((None, tn, cout), lambda bi, ni: (bi, ni, 0)),
            pl.BlockSpec((None, tn, cout), lambda bi, ni: (bi, ni, 0)),
        ),
        compiler_params=pltpu.CompilerParams(
            dimension_semantics=("parallel", "parallel")),
    )(x, w_cat)


def _edge_kernel(idx_ref, p_ref, c_ref, b_ref, o_ref, idx_smem, sem,
                 *, k, pu):
    cp = pltpu.make_async_copy(idx_ref, idx_smem, sem)
    cp.start()
    cp.wait()
    bias = b_ref[0]
    tn = c_ref.shape[0]

    def body(oi, carry):
        base = oi * pu
        for pi in range(pu):
            i = base + pi
            m = p_ref[idx_smem[0, 0, i * k]]
            for kk in range(1, k):
                m = jnp.maximum(m, p_ref[idx_smem[0, 0, i * k + kk]])
            y = m + c_ref[i] + bias
            o_ref[i] = _leaky(y)
        return carry

    jax.lax.fori_loop(0, tn // pu, body, 0)


def _edge(p, c, idx, bias, tn, k=_KNN, pu=8):
    b, n, cout = c.shape
    nb = n // tn
    p4 = p.reshape(b, n, 1, cout)
    c4 = c.reshape(b, n, 1, cout)
    idx3 = idx.reshape(b * nb, 1, tn * k)
    b3 = bias.reshape(1, 1, cout)
    out = pl.pallas_call(
        functools.partial(_edge_kernel, k=k, pu=pu),
        out_shape=jax.ShapeDtypeStruct((b, n, 1, cout), jnp.float32),
        grid=(b, nb),
        in_specs=[
            pl.BlockSpec((1, 1, tn * k), lambda bi, ni, nb=nb: (bi * nb + ni, 0, 0)),
            pl.BlockSpec((None, n, 1, cout), lambda bi, ni: (bi, 0, 0, 0)),
            pl.BlockSpec((None, tn, 1, cout), lambda bi, ni: (bi, ni, 0, 0)),
            pl.BlockSpec((1, 1, cout), lambda bi, ni: (0, 0, 0)),
        ],
        out_specs=pl.BlockSpec((None, tn, 1, cout),
                               lambda bi, ni: (bi, ni, 0, 0)),
        scratch_shapes=[
            pltpu.SMEM((1, 1, tn * k), jnp.int32),
            pltpu.SemaphoreType.DMA,
        ],
        compiler_params=pltpu.CompilerParams(
            dimension_semantics=("parallel", "parallel")),
    )(idx3, p4, c4, b3)
    return out.reshape(b, n, cout)


def _head_kernel(x1_ref, x2_ref, x3_ref, x4_ref,
                 w1_ref, w2_ref, w3_ref, w4_ref, b_ref,
                 y_ref, idx_ref, pool_ref, *, te):
    e0 = pl.multiple_of(pl.program_id(1) * te, te)
    y = jnp.dot(w1_ref[pl.ds(e0, te), :], x1_ref[...],
                preferred_element_type=jnp.float32)
    y = y + jnp.dot(w2_ref[pl.ds(e0, te), :], x2_ref[...],
                    preferred_element_type=jnp.float32)
    y = y + jnp.dot(w3_ref[pl.ds(e0, te), :], x3_ref[...],
                    preferred_element_type=jnp.float32)
    y = y + jnp.dot(w4_ref[pl.ds(e0, te), :], x4_ref[...],
                    preferred_element_type=jnp.float32)
    y = y + b_ref[...]
    y = _leaky(y)
    y_ref[...] = y

    mx = jnp.max(y, axis=1, keepdims=True)
    cols = jax.lax.broadcasted_iota(jnp.int32, y.shape, 1)
    amin = jnp.min(jnp.where(y == mx, cols, jnp.int32(2 ** 30)),
                   axis=1, keepdims=True)
    pool_ref[...] = mx
    idx_ref[...] = amin


def _head(x1t, x2t, x3t, x4t, w5, scale, bias, te):
    b = x1t.shape[0]
    n = x1t.shape[-1]
    emb = w5.shape[1]
    c1, c2, c3, c4 = (x1t.shape[1], x2t.shape[1], x3t.shape[1], x4t.shape[1])
    w5t = (w5 * scale).T
    w_1 = w5t[:, :c1]
    w_2 = w5t[:, c1:c1 + c2]
    w_3 = w5t[:, c1 + c2:c1 + c2 + c3]
    w_4 = w5t[:, c1 + c2 + c3:]
    bt = bias.reshape(emb, 1)
    return pl.pallas_call(
        functools.partial(_head_kernel, te=te),
        out_shape=(
            jax.ShapeDtypeStruct((b, emb, n), jnp.float32),
            jax.ShapeDtypeStruct((b, emb, 1), jnp.int32),
            jax.ShapeDtypeStruct((b, emb, 1), jnp.float32),
        ),
        grid=(b, emb // te),
        in_specs=[
            pl.BlockSpec((None, c1, n), lambda bi, ei: (bi, 0, 0)),
            pl.BlockSpec((None, c2, n), lambda bi, ei: (bi, 0, 0)),
            pl.BlockSpec((None, c3, n), lambda bi, ei: (bi, 0, 0)),
            pl.BlockSpec((None, c4, n), lambda bi, ei: (bi, 0, 0)),
            pl.BlockSpec((emb, c1), lambda bi, ei: (0, 0)),
            pl.BlockSpec((emb, c2), lambda bi, ei: (0, 0)),
            pl.BlockSpec((emb, c3), lambda bi, ei: (0, 0)),
            pl.BlockSpec((emb, c4), lambda bi, ei: (0, 0)),
            pl.BlockSpec((te, 1), lambda bi, ei: (ei, 0)),
        ],
        out_specs=(
            pl.BlockSpec((None, te, n), lambda bi, ei: (bi, ei, 0)),
            pl.BlockSpec((None, te, 1), lambda bi, ei: (bi, ei, 0)),
            pl.BlockSpec((None, te, 1), lambda bi, ei: (bi, ei, 0)),
        ),
        compiler_params=pltpu.CompilerParams(
            dimension_semantics=("parallel", "parallel")),
    )(x1t, x2t, x3t, x4t, w_1, w_2, w_3, w_4, bt)


def _knn_idx(x, k):
    xx = jnp.sum(x * x, axis=-1)
    inner = jnp.einsum("bnc,bmc->bnm", x, x)
    pairwise = -xx[:, :, None] + 2.0 * inner - xx[:, None, :]
    _, idx = jax.lax.top_k(pairwise, k)
    return idx


def _edge_conv(x, w, scale, bias, tn):
    cin = x.shape[-1]
    idx = _knn_idx(x, _KNN)
    w_nbr = w[:cin] * scale
    w_dif = (w[cin:] - w[:cin]) * scale
    w_cat = jnp.concatenate([w_nbr, w_dif], axis=1)
    p, c = _point_conv(x, w_cat, tn)
    return _edge(p, c, idx, bias, tn)


def kernel(x_bcn, w1, w2, w3, w4, w5,
           bn1_scale, bn1_bias, bn2_scale, bn2_bias,
           bn3_scale, bn3_bias, bn4_scale, bn4_bias,
           bn5_scale, bn5_bias):
    x = jnp.transpose(x_bcn, (0, 2, 1)).astype(jnp.float32)
    n = x.shape[1]
    tn = _pick_tile(n)

    x1 = _edge_conv(x, w1, bn1_scale, bn1_bias, tn)
    x2 = _edge_conv(x1, w2, bn2_scale, bn2_bias, tn)
    x3 = _edge_conv(x2, w3, bn3_scale, bn3_bias, tn)
    x4 = _edge_conv(x3, w4, bn4_scale, bn4_bias, tn)

    x1t = jnp.transpose(x1, (0, 2, 1))
    x2t = jnp.transpose(x2, (0, 2, 1))
    x3t = jnp.transpose(x3, (0, 2, 1))
    x4t = jnp.transpose(x4, (0, 2, 1))

    emb = w5.shape[1]
    te = 512 if emb % 512 == 0 else _pick_tile(emb)
    y, idx, pool = _head(x1t, x2t, x3t, x4t, w5, bn5_scale, bn5_bias, te)
    return y, idx[:, :, 0], pool[:, :, 0]


# speedup vs baseline: 7.5063x; 2.6295x over previous
import functools

import jax
import jax.numpy as jnp
from jax.experimental import pallas as pl
from jax.experimental.pallas import tpu as pltpu

_NEG_SLOPE = 0.2
_KNN = 20


def _pick_tile(n, target=256):
    if n <= target:
        return n
    for t in range(target, 7, -1):
        if n % t == 0 and t % 8 == 0:
            return t
    return n


def _leaky(v):
    return jnp.where(v > 0, v, _NEG_SLOPE * v)


def _pc_kernel(x_ref, w_ref, p_ref, c_ref):
    y = jnp.dot(x_ref[...], w_ref[...], preferred_element_type=jnp.float32)
    cout = p_ref.shape[-1]
    p_ref[...] = y[:, :cout]
    c_ref[...] = y[:, cout:]


def _point_conv(x, w_cat, tn):
    b, n, cin = x.shape
    cout = w_cat.shape[1] // 2
    return pl.pallas_call(
        _pc_kernel,
        out_shape=(
            jax.ShapeDtypeStruct((b, n, cout), jnp.float32),
            jax.ShapeDtypeStruct((b, n, cout), jnp.float32),
        ),
        grid=(b, n // tn),
        in_specs=[
            pl.BlockSpec((None, tn, cin), lambda bi, ni: (bi, ni, 0)),
            pl.BlockSpec((cin, 2 * cout), lambda bi, ni: (0, 0)),
        ],
        out_specs=(
            pl.BlockSpec((None, tn, cout), lambda bi, ni: (bi, ni, 0)),
            pl.BlockSpec((None, tn, cout), lambda bi, ni: (bi, ni, 0)),
        ),
        compiler_params=pltpu.CompilerParams(
            dimension_semantics=("parallel", "parallel")),
    )(x, w_cat)


def _edge_kernel(idx_ref, p_ref, c_ref, b_ref, o_ref, idx_smem, sem,
                 *, k, pu):
    cp = pltpu.make_async_copy(idx_ref, idx_smem, sem)
    cp.start()
    cp.wait()
    bias = b_ref[0]
    tn = c_ref.shape[0]

    def body(oi, carry):
        base = oi * pu
        for pi in range(pu):
            i = base + pi
            m = p_ref[idx_smem[0, 0, i * k]]
            for kk in range(1, k):
                m = jnp.maximum(m, p_ref[idx_smem[0, 0, i * k + kk]])
            y = m + c_ref[i] + bias
            o_ref[i] = _leaky(y)
        return carry

    jax.lax.fori_loop(0, tn // pu, body, 0)


def _edge(p, c, idx, bias, tn, k=_KNN, pu=8):
    b, n, cout = c.shape
    nb = n // tn
    p4 = p.reshape(b, n, 1, cout)
    c4 = c.reshape(b, n, 1, cout)
    idx3 = idx.reshape(b * nb, 1, tn * k)
    b3 = bias.reshape(1, 1, cout)
    out = pl.pallas_call(
        functools.partial(_edge_kernel, k=k, pu=pu),
        out_shape=jax.ShapeDtypeStruct((b, n, 1, cout), jnp.float32),
        grid=(b, nb),
        in_specs=[
            pl.BlockSpec((1, 1, tn * k), lambda bi, ni, nb=nb: (bi * nb + ni, 0, 0)),
            pl.BlockSpec((None, n, 1, cout), lambda bi, ni: (bi, 0, 0, 0)),
            pl.BlockSpec((None, tn, 1, cout), lambda bi, ni: (bi, ni, 0, 0)),
            pl.BlockSpec((1, 1, cout), lambda bi, ni: (0, 0, 0)),
        ],
        out_specs=pl.BlockSpec((None, tn, 1, cout),
                               lambda bi, ni: (bi, ni, 0, 0)),
        scratch_shapes=[
            pltpu.SMEM((1, 1, tn * k), jnp.int32),
            pltpu.SemaphoreType.DMA,
        ],
        compiler_params=pltpu.CompilerParams(
            dimension_semantics=("parallel", "parallel")),
    )(idx3, p4, c4, b3)
    return out.reshape(b, n, cout)


def _head_kernel(x1_ref, x2_ref, x3_ref, x4_ref,
                 w1_ref, w2_ref, w3_ref, w4_ref, b_ref,
                 y_ref, idx_ref, pool_ref, *, te):
    e0 = pl.multiple_of(pl.program_id(1) * te, te)
    y = jnp.dot(w1_ref[pl.ds(e0, te), :], x1_ref[...],
                preferred_element_type=jnp.float32)
    y = y + jnp.dot(w2_ref[pl.ds(e0, te), :], x2_ref[...],
                    preferred_element_type=jnp.float32)
    y = y + jnp.dot(w3_ref[pl.ds(e0, te), :], x3_ref[...],
                    preferred_element_type=jnp.float32)
    y = y + jnp.dot(w4_ref[pl.ds(e0, te), :], x4_ref[...],
                    preferred_element_type=jnp.float32)
    y = y + b_ref[...]
    y = _leaky(y)
    y_ref[...] = y

    mx = jnp.max(y, axis=1, keepdims=True)
    cols = jax.lax.broadcasted_iota(jnp.int32, y.shape, 1)
    amin = jnp.min(jnp.where(y == mx, cols, jnp.int32(2 ** 30)),
                   axis=1, keepdims=True)
    pool_ref[...] = mx
    idx_ref[...] = amin


def _head(x1t, x2t, x3t, x4t, w5, scale, bias, te):
    b = x1t.shape[0]
    n = x1t.shape[-1]
    emb = w5.shape[1]
    c1, c2, c3, c4 = (x1t.shape[1], x2t.shape[1], x3t.shape[1], x4t.shape[1])
    w5t = (w5 * scale).T
    w_1 = w5t[:, :c1]
    w_2 = w5t[:, c1:c1 + c2]
    w_3 = w5t[:, c1 + c2:c1 + c2 + c3]
    w_4 = w5t[:, c1 + c2 + c3:]
    bt = bias.reshape(emb, 1)
    return pl.pallas_call(
        functools.partial(_head_kernel, te=te),
        out_shape=(
            jax.ShapeDtypeStruct((b, emb, n), jnp.float32),
            jax.ShapeDtypeStruct((b, emb, 1), jnp.int32),
            jax.ShapeDtypeStruct((b, emb, 1), jnp.float32),
        ),
        grid=(b, emb // te),
        in_specs=[
            pl.BlockSpec((None, c1, n), lambda bi, ei: (bi, 0, 0)),
            pl.BlockSpec((None, c2, n), lambda bi, ei: (bi, 0, 0)),
            pl.BlockSpec((None, c3, n), lambda bi, ei: (bi, 0, 0)),
            pl.BlockSpec((None, c4, n), lambda bi, ei: (bi, 0, 0)),
            pl.BlockSpec((emb, c1), lambda bi, ei: (0, 0)),
            pl.BlockSpec((emb, c2), lambda bi, ei: (0, 0)),
            pl.BlockSpec((emb, c3), lambda bi, ei: (0, 0)),
            pl.BlockSpec((emb, c4), lambda bi, ei: (0, 0)),
            pl.BlockSpec((te, 1), lambda bi, ei: (ei, 0)),
        ],
        out_specs=(
            pl.BlockSpec((None, te, n), lambda bi, ei: (bi, ei, 0)),
            pl.BlockSpec((None, te, 1), lambda bi, ei: (bi, ei, 0)),
            pl.BlockSpec((None, te, 1), lambda bi, ei: (bi, ei, 0)),
        ),
        compiler_params=pltpu.CompilerParams(
            dimension_semantics=("parallel", "parallel")),
    )(x1t, x2t, x3t, x4t, w_1, w_2, w_3, w_4, bt)


def _knn_kernel(x_ref, xt_ref, xxc_ref, xxr_ref, idx_ref, s_ref, *, k):
    s = jnp.dot(x_ref[...], xt_ref[...], preferred_element_type=jnp.float32)
    s_ref[...] = (-xxr_ref[...] + 2.0 * s) - xxc_ref[...]
    n, tr = s_ref.shape
    cand = jax.lax.broadcasted_iota(jnp.int32, (n, tr), 0)
    kiota = jax.lax.broadcasted_iota(jnp.int32, (k, tr), 0)
    big = jnp.int32(2 ** 30)

    def round_body(r, p):
        sv = s_ref[...]
        m = jnp.max(sv, axis=0, keepdims=True)
        pick = jnp.min(jnp.where(sv == m, cand, big), axis=0, keepdims=True)
        s_ref[...] = jnp.where(cand == pick, -jnp.inf, sv)
        return jnp.where(kiota == r, pick, p)

    idx_ref[...] = jax.lax.fori_loop(
        0, k, round_body, jnp.zeros((k, tr), jnp.int32))


def _knn_idx(x, xt, k, tr=256):
    b, n, c = x.shape
    tr = min(tr, n)
    xx = jnp.sum(x * x, axis=-1)
    xxc = xx[:, :, None]
    xxr = xx[:, None, :]
    idx_kn = pl.pallas_call(
        functools.partial(_knn_kernel, k=k),
        out_shape=jax.ShapeDtypeStruct((b, k, n), jnp.int32),
        grid=(b, n // tr),
        in_specs=[
            pl.BlockSpec((None, n, c), lambda bi, ri: (bi, 0, 0)),
            pl.BlockSpec((None, c, tr), lambda bi, ri: (bi, 0, ri)),
            pl.BlockSpec((None, n, 1), lambda bi, ri: (bi, 0, 0)),
            pl.BlockSpec((None, 1, tr), lambda bi, ri: (bi, 0, ri)),
        ],
        out_specs=pl.BlockSpec((None, k, tr), lambda bi, ri: (bi, 0, ri)),
        scratch_shapes=[pltpu.VMEM((n, tr), jnp.float32)],
        compiler_params=pltpu.CompilerParams(
            dimension_semantics=("parallel", "parallel")),
    )(x, xt, xxc, xxr)
    return jnp.transpose(idx_kn, (0, 2, 1))


def _edge_conv(x, xt, w, scale, bias, tn):
    cin = x.shape[-1]
    idx = _knn_idx(x, xt, _KNN)
    w_nbr = w[:cin] * scale
    w_dif = (w[cin:] - w[:cin]) * scale
    w_cat = jnp.concatenate([w_nbr, w_dif], axis=1)
    p, c = _point_conv(x, w_cat, tn)
    return _edge(p, c, idx, bias, tn)


def kernel(x_bcn, w1, w2, w3, w4, w5,
           bn1_scale, bn1_bias, bn2_scale, bn2_bias,
           bn3_scale, bn3_bias, bn4_scale, bn4_bias,
           bn5_scale, bn5_bias):
    xt0 = x_bcn.astype(jnp.float32)
    x = jnp.transpose(xt0, (0, 2, 1))
    n = x.shape[1]
    tn = _pick_tile(n)

    x1 = _edge_conv(x, xt0, w1, bn1_scale, bn1_bias, tn)
    x1t = jnp.transpose(x1, (0, 2, 1))
    x2 = _edge_conv(x1, x1t, w2, bn2_scale, bn2_bias, tn)
    x2t = jnp.transpose(x2, (0, 2, 1))
    x3 = _edge_conv(x2, x2t, w3, bn3_scale, bn3_bias, tn)
    x3t = jnp.transpose(x3, (0, 2, 1))
    x4 = _edge_conv(x3, x3t, w4, bn4_scale, bn4_bias, tn)
    x4t = jnp.transpose(x4, (0, 2, 1))

    emb = w5.shape[1]
    te = 512 if emb % 512 == 0 else _pick_tile(emb)
    y, idx, pool = _head(x1t, x2t, x3t, x4t, w5, bn5_scale, bn5_bias, te)
    return y, idx[:, :, 0], pool[:, :, 0]
